# Initial kernel scaffold; baseline (speedup 1.0000x reference)
import jax
import jax.numpy as jnp
from jax.experimental import pallas as pl


def kernel(x, input_emb):
    raise NotImplementedError("write your pallas kernel here")


# speedup vs baseline: 1.0940x; 1.0940x over previous
import functools

import jax
import jax.numpy as jnp
from jax import lax
from jax.experimental import pallas as pl
from jax.experimental.pallas import tpu as pltpu
from jax.experimental.pallas import tpu_sc as plsc

_VOCAB = 1000000
_EMB = 32
_B = 16384
_L = 50
_N = _B * _L

_NC = 2
_NS = 16
_NW = _NC * _NS
_PW = _N // _NW
_C = 1024
_NCHUNK = _PW // _C

_mesh = plsc.VectorSubcoreMesh(core_axis_name="c", subcore_axis_name="s")


@functools.partial(
    pl.kernel,
    out_type=jax.ShapeDtypeStruct((_N, _EMB), jnp.float32),
    mesh=_mesh,
    scratch_types=[
        pltpu.VMEM((_C,), jnp.int32),
        pltpu.VMEM((_C, _EMB), jnp.float32),
        pltpu.SemaphoreType.DMA,
    ],
    compiler_params=pltpu.CompilerParams(use_tc_tiling_on_sc=False),
)
def _gather(idx_hbm, table_hbm, out_hbm, idx_v, rows_v, sem):
    wid = lax.axis_index("s") * _NC + lax.axis_index("c")
    base = wid * _PW

    @pl.loop(0, _NCHUNK)
    def _chunk(g):
        off = base + g * _C
        pltpu.sync_copy(idx_hbm.at[pl.ds(off, _C)], idx_v)
        pltpu.async_copy(table_hbm.at[idx_v], rows_v, sem).wait()
        pltpu.sync_copy(rows_v, out_hbm.at[pl.ds(off, _C)])


def kernel(x, input_emb):
    out = _gather(x.reshape(_N), input_emb)
    return out.reshape(_B, _L, _EMB)


# speedup vs baseline: 1.1125x; 1.0169x over previous
import functools

import jax
import jax.numpy as jnp
from jax import lax
from jax.experimental import pallas as pl
from jax.experimental.pallas import tpu as pltpu
from jax.experimental.pallas import tpu_sc as plsc

_VOCAB = 1000000
_EMB = 32
_B = 16384
_L = 50
_N = _B * _L

_NC = 2
_NS = 16
_NW = _NC * _NS
_PW = _N // _NW
_C = 1600
_NCHUNK = _PW // _C

_mesh = plsc.VectorSubcoreMesh(core_axis_name="c", subcore_axis_name="s")


@functools.partial(
    pl.kernel,
    out_type=jax.ShapeDtypeStruct((_N, _EMB), jnp.float32),
    mesh=_mesh,
    scratch_types=[
        pltpu.VMEM((_PW,), jnp.int32),
        pltpu.VMEM((_C, _EMB), jnp.float32),
        pltpu.VMEM((_C, _EMB), jnp.float32),
        pltpu.SemaphoreType.DMA,
        pltpu.SemaphoreType.DMA,
    ],
    compiler_params=pltpu.CompilerParams(use_tc_tiling_on_sc=False),
)
def _gather(idx_hbm, table_hbm, out_hbm, idx_v, rows0, rows1, sem0, sem1):
    wid = lax.axis_index("s") * _NC + lax.axis_index("c")
    base = wid * _PW
    rows = (rows0, rows1)
    sems = (sem0, sem1)

    pltpu.sync_copy(idx_hbm.at[pl.ds(base, _PW)], idx_v)

    def _desc(c, b):
        return pltpu.make_async_copy(
            table_hbm.at[idx_v.at[pl.ds(c * _C, _C)]], rows[b], sems[b]
        )

    _desc(0, 0).start()

    @pl.loop(0, _NCHUNK, step=2)
    def _grp(g):
        for b in range(2):
            c = g + b

            @pl.when(c + 1 < _NCHUNK)
            def _():
                _desc(c + 1, 1 - b).start()

            _desc(c, b).wait()
            pltpu.sync_copy(rows[b], out_hbm.at[pl.ds(base + c * _C, _C)])


def kernel(x, input_emb):
    out = _gather(x.reshape(_N), input_emb)
    return out.reshape(_B, _L, _EMB)


# speedup vs baseline: 1.8085x; 1.6257x over previous
import functools

import jax
import jax.numpy as jnp
from jax import lax
from jax.experimental import pallas as pl
from jax.experimental.pallas import tpu as pltpu
from jax.experimental.pallas import tpu_sc as plsc

_VOCAB = 1000000
_EMB = 32
_B = 16384
_L = 50
_N = _B * _L

_NC = 2
_NS = 16
_NW = _NC * _NS
_PW = _N // _NW
_C = 1600
_NCHUNK = _PW // _C
_CB = _C // _L

_mesh = plsc.VectorSubcoreMesh(core_axis_name="c", subcore_axis_name="s")


@functools.partial(
    pl.kernel,
    out_type=jax.ShapeDtypeStruct((_B, _L, _EMB), jnp.float32),
    mesh=_mesh,
    scratch_types=[
        pltpu.VMEM((_PW,), jnp.int32),
        pltpu.VMEM((_C, _EMB), jnp.float32),
        pltpu.VMEM((_C, _EMB), jnp.float32),
        pltpu.SemaphoreType.DMA,
        pltpu.SemaphoreType.DMA,
        pltpu.SemaphoreType.DMA,
        pltpu.SemaphoreType.DMA,
    ],
    compiler_params=pltpu.CompilerParams(use_tc_tiling_on_sc=False),
)
def _gather(idx_hbm, table_hbm, out_3d, idx_v, rows0, rows1,
            gsem0, gsem1, osem0, osem1):
    wid = lax.axis_index("s") * _NC + lax.axis_index("c")
    base = wid * _PW
    bbase = wid * (_PW // _L)
    rows = (rows0, rows1)
    gsems = (gsem0, gsem1)
    osems = (osem0, osem1)

    pltpu.sync_copy(idx_hbm.at[pl.ds(base, _PW)], idx_v)

    def _gat(c, b):
        return pltpu.make_async_copy(
            table_hbm.at[idx_v.at[pl.ds(c * _C, _C)]], rows[b], gsems[b]
        )

    def _out(c, b, i):
        return pltpu.make_async_copy(
            rows[b].at[pl.ds(i * _L, _L)],
            out_3d.at[bbase + c * _CB + i],
            osems[b],
        )

    _gat(0, 0).start()

    @pl.loop(0, _NCHUNK, step=2)
    def _grp(g):
        for b in range(2):
            c = g + b

            @pl.when(c >= 1)
            def _():
                for i in range(_CB):
                    _out(c - 1, 1 - b, i).wait()

            @pl.when(c + 1 < _NCHUNK)
            def _():
                _gat(c + 1, 1 - b).start()

            _gat(c, b).wait()
            for i in range(_CB):
                _out(c, b, i).start()

    for i in range(_CB):
        _out(_NCHUNK - 1, 1, i).wait()


def kernel(x, input_emb):
    return _gather(x.reshape(_N), input_emb)
